# baseline (speedup 1.0000x reference)
import functools

import jax
import jax.numpy as jnp
from jax import lax
from jax.experimental import pallas as pl
from jax.experimental.pallas import tpu as pltpu
from jax.experimental.pallas import tpu_sc as plsc

NC = 2
NS = 16
NW = NC * NS
B = 128


def _sc_aggregate(n_pad, d, k_chunks, with_counts):
  rows_per = n_pad // NS

  mesh = plsc.VectorSubcoreMesh(core_axis_name="c", subcore_axis_name="s")

  @functools.partial(
      pl.kernel,
      mesh=mesh,
      compiler_params=pltpu.CompilerParams(use_tc_tiling_on_sc=False),
      out_type=[
          jax.ShapeDtypeStruct((NC, n_pad, d), jnp.float32),
          jax.ShapeDtypeStruct((NC * n_pad,), jnp.float32),
      ],
      scratch_types=[
          pltpu.VMEM((2, B), jnp.int32),
          pltpu.VMEM((4, B), jnp.int32),
          pltpu.VMEM((2, B, d), jnp.float32),
          pltpu.VMEM((B,), jnp.float32),
          pltpu.VMEM((rows_per,), jnp.float32),
          pltpu.VMEM_SHARED((n_pad, d), jnp.float32),
          pltpu.VMEM_SHARED((n_pad,), jnp.float32),
          pltpu.SemaphoreType.DMA,
          pltpu.SemaphoreType.DMA,
          pltpu.SemaphoreType.DMA,
          pltpu.SemaphoreType.DMA,
          pltpu.SemaphoreType.DMA,
          pltpu.SemaphoreType.DMA,
          pltpu.SemaphoreType.DMA,
      ],
  )
  def agg(table_hbm, src_hbm, dst_hbm, z2_hbm, z1_hbm, ones_hbm,
          psum_out, cnt_out,
          src_v, dst_v, rows_v, ones_v, cnt_v, accum, cnt_acc,
          gsem0, gsem1, ssem0, ssem1, isem0, isem1, csem):
    c = lax.axis_index("c")
    s = lax.axis_index("s")
    r0 = s * rows_per
    gsem = (gsem0, gsem1)
    ssem = (ssem0, ssem1)
    isem = (isem0, isem1)

    pltpu.sync_copy(z2_hbm.at[pl.ds(r0, rows_per)],
                    accum.at[pl.ds(r0, rows_per)])
    if with_counts:
      pltpu.sync_copy(z1_hbm.at[pl.ds(r0, rows_per)], cnt_v)
      pltpu.sync_copy(cnt_v, cnt_acc.at[pl.ds(r0, rows_per)])
      pltpu.sync_copy(ones_hbm, ones_v)
    plsc.subcore_barrier()

    def fetch_idx(k, k2, k4):
      pltpu.async_copy(src_hbm.at[c, s, k], src_v.at[k2], isem[k2])
      pltpu.async_copy(dst_hbm.at[c, s, k], dst_v.at[k4], isem[k2])

    def drain_idx(k, k2, k4):
      pltpu.make_async_copy(src_hbm.at[c, s, k], src_v.at[k2],
                            isem[k2]).wait()
      pltpu.make_async_copy(dst_hbm.at[c, s, k], dst_v.at[k4],
                            isem[k2]).wait()

    def fire_gather(k2):
      pltpu.async_copy(table_hbm.at[src_v.at[k2]], rows_v.at[k2], gsem[k2])

    def drain_gather(k2):
      pltpu.make_async_copy(table_hbm.at[src_v.at[k2]], rows_v.at[k2],
                            gsem[k2]).wait()

    def fire_scatter(k2, k4):
      pltpu.async_copy(rows_v.at[k2], accum.at[dst_v.at[k4]],
                       ssem[k2], add=True)

    def drain_scatter(k2, k4):
      pltpu.make_async_copy(rows_v.at[k2], accum.at[dst_v.at[k4]],
                            ssem[k2]).wait()

    fetch_idx(0, 0, 0)
    drain_idx(0, 0, 0)
    fetch_idx(1, 1, 1)
    fire_gather(0)

    def body(i4, carry):
      for q in range(4):
        k = i4 * 4 + q
        p2, p4 = q % 2, q
        drain_gather(p2)
        fire_scatter(p2, p4)
        if with_counts:
          pltpu.async_copy(ones_v, cnt_acc.at[dst_v.at[p4]], csem, add=True)

        @pl.when(k + 1 < k_chunks)
        def _():
          drain_idx(k + 1, 1 - p2, (q + 1) % 4)

        @pl.when(k >= 1)
        def _():
          drain_scatter(1 - p2, (q + 3) % 4)

        @pl.when(k + 1 < k_chunks)
        def _():
          fire_gather(1 - p2)

        @pl.when(k + 2 < k_chunks)
        def _():
          fetch_idx(k + 2, p2, (q + 2) % 4)

        if with_counts:
          pltpu.make_async_copy(ones_v, cnt_acc.at[dst_v.at[p4]],
                                csem).wait()
      return carry

    lax.fori_loop(0, k_chunks // 4, body, 0)
    drain_scatter((k_chunks - 1) % 2, (k_chunks - 1) % 4)
    plsc.subcore_barrier()

    pltpu.sync_copy(accum.at[pl.ds(r0, rows_per)],
                    psum_out.at[c, pl.ds(r0, rows_per)])
    if with_counts:
      pltpu.sync_copy(cnt_acc.at[pl.ds(r0, rows_per)], cnt_v)
      pltpu.sync_copy(cnt_v, cnt_out.at[pl.ds(c * n_pad + r0, rows_per)])

  return agg


def _tc_layer(n, n_pad, d, relu):
  r = 2000
  dot = functools.partial(
      lax.dot_general,
      dimension_numbers=(((1,), (1,)), ((), ())),
      preferred_element_type=jnp.float32,
  )

  def body(x_ref, p_ref, c_ref, wl_ref, wr_ref, b_ref, o_ref):
    cnt = jnp.maximum(c_ref[0] + c_ref[1], 1.0)
    mean = (p_ref[0] + p_ref[1]) / cnt
    h = dot(mean, wl_ref[...]) + dot(x_ref[...], wr_ref[...]) + b_ref[...]
    if relu:
      h = jnp.maximum(h, 0.0)
    o_ref[...] = h

  return pl.pallas_call(
      body,
      grid=(n // r,),
      in_specs=[
          pl.BlockSpec((r, d), lambda i: (i, 0)),
          pl.BlockSpec((NC, r, d), lambda i: (0, i, 0)),
          pl.BlockSpec((NC, r, 1), lambda i: (0, i, 0)),
          pl.BlockSpec((d, d), lambda i: (0, 0)),
          pl.BlockSpec((d, d), lambda i: (0, 0)),
          pl.BlockSpec((1, d), lambda i: (0, 0)),
      ],
      out_specs=pl.BlockSpec((r, d), lambda i: (i, 0)),
      out_shape=jax.ShapeDtypeStruct((n, d), jnp.float32),
  )


def kernel(x, edge_index, W1_l, b1, W1_r, W2_l, b2, W2_r):
  n, d = x.shape
  e = edge_index.shape[1]

  k_chunks = -(-e // (NW * B * 4)) * 4
  e_pad = NW * B * k_chunks
  n_pad = -(-(n + 1) // (NS * 8)) * (NS * 8)

  src = edge_index[0]
  dst = edge_index[1]
  pad = e_pad - e
  src4 = jnp.concatenate([src, jnp.zeros((pad,), jnp.int32)]).reshape(
      NC, NS, k_chunks, B)
  dst4 = jnp.concatenate([dst, jnp.full((pad,), n, jnp.int32)]).reshape(
      NC, NS, k_chunks, B)

  z2 = jnp.zeros((n_pad, d), jnp.float32)
  z1 = jnp.zeros((n_pad,), jnp.float32)
  ones = jnp.ones((B,), jnp.float32)

  agg1 = _sc_aggregate(n_pad, d, k_chunks, with_counts=True)
  agg2 = _sc_aggregate(n_pad, d, k_chunks, with_counts=False)
  l1 = _tc_layer(n, n_pad, d, relu=True)
  l2 = _tc_layer(n, n_pad, d, relu=False)

  b1r = b1.reshape(1, d)
  b2r = b2.reshape(1, d)

  p1, c1 = agg1(x, src4, dst4, z2, z1, ones)
  c1r = c1.reshape(NC, n_pad, 1)
  h = l1(x, p1, c1r, W1_l, W1_r, b1r)
  p2, _ = agg2(h, src4, dst4, z2, z1, ones)
  out = l2(h, p2, c1r, W2_l, W2_r, b2r)
  return out


# speedup vs baseline: 1.8874x; 1.8874x over previous
import functools

import jax
import jax.numpy as jnp
from jax import lax
from jax.experimental import pallas as pl
from jax.experimental.pallas import tpu as pltpu
from jax.experimental.pallas import tpu_sc as plsc

NC = 2
NS = 16
B = 128


def _sc_aggregate(n_pad, hd, k_chunks, with_counts):
  rows_per = n_pad // NS

  mesh = plsc.VectorSubcoreMesh(core_axis_name="c", subcore_axis_name="s")

  @functools.partial(
      pl.kernel,
      mesh=mesh,
      compiler_params=pltpu.CompilerParams(use_tc_tiling_on_sc=False),
      out_type=[
          jax.ShapeDtypeStruct((NC, n_pad, hd), jnp.float32),
          jax.ShapeDtypeStruct((n_pad,), jnp.float32),
      ],
      scratch_types=[
          pltpu.VMEM((k_chunks, B), jnp.int32),
          pltpu.VMEM((k_chunks, B), jnp.int32),
          pltpu.VMEM((2, B, hd), jnp.float32),
          pltpu.VMEM((B,), jnp.float32),
          pltpu.VMEM((rows_per,), jnp.float32),
          pltpu.VMEM_SHARED((n_pad, hd), jnp.float32),
          pltpu.VMEM_SHARED((n_pad,), jnp.float32),
          pltpu.SemaphoreType.DMA,
          pltpu.SemaphoreType.DMA,
          pltpu.SemaphoreType.DMA,
          pltpu.SemaphoreType.DMA,
          pltpu.SemaphoreType.DMA,
      ],
  )
  def agg(table_hbm, src_hbm, dst_hbm, z2_hbm, z1_hbm, ones_hbm,
          psum_out, cnt_out,
          src_all, dst_all, rows_v, ones_v, cnt_v, accum, cnt_acc,
          gsem0, gsem1, ssem0, ssem1, csem):
    c = lax.axis_index("c")
    s = lax.axis_index("s")
    r0 = s * rows_per
    gsem = (gsem0, gsem1)
    ssem = (ssem0, ssem1)
    half = table_hbm.at[c]

    pltpu.sync_copy(src_hbm.at[s], src_all)
    pltpu.sync_copy(dst_hbm.at[s], dst_all)
    pltpu.sync_copy(z2_hbm.at[pl.ds(r0, rows_per)],
                    accum.at[pl.ds(r0, rows_per)])
    if with_counts:
      @pl.when(c == 0)
      def _():
        pltpu.sync_copy(z1_hbm.at[pl.ds(r0, rows_per)], cnt_v)
        pltpu.sync_copy(cnt_v, cnt_acc.at[pl.ds(r0, rows_per)])
        pltpu.sync_copy(ones_hbm, ones_v)
    plsc.subcore_barrier()

    def fire_gather(k, par):
      pltpu.async_copy(half.at[src_all.at[k]], rows_v.at[par], gsem[par])

    def drain_gather(k, par):
      pltpu.make_async_copy(half.at[src_all.at[k]], rows_v.at[par],
                            gsem[par]).wait()

    def fire_scatter(k, par):
      pltpu.async_copy(rows_v.at[par], accum.at[dst_all.at[k]],
                       ssem[par], add=True)

    def drain_scatter(k, par):
      pltpu.make_async_copy(rows_v.at[par], accum.at[dst_all.at[k]],
                            ssem[par]).wait()

    fire_gather(0, 0)

    def body(i2, carry):
      for par in range(2):
        k = i2 * 2 + par
        drain_gather(k, par)
        fire_scatter(k, par)
        if with_counts:
          @pl.when(c == 0)
          def _():
            pltpu.async_copy(ones_v, cnt_acc.at[dst_all.at[k]],
                             csem, add=True)

        @pl.when(k + 1 < k_chunks)
        def _():
          @pl.when(k >= 1)
          def _():
            drain_scatter(k - 1, 1 - par)
          fire_gather(k + 1, 1 - par)

        if with_counts:
          @pl.when(c == 0)
          def _():
            pltpu.make_async_copy(ones_v, cnt_acc.at[dst_all.at[k]],
                                  csem).wait()
      return carry

    lax.fori_loop(0, k_chunks // 2, body, 0)
    drain_scatter(k_chunks - 1, 1)
    plsc.subcore_barrier()

    pltpu.sync_copy(accum.at[pl.ds(r0, rows_per)],
                    psum_out.at[c, pl.ds(r0, rows_per)])
    if with_counts:
      @pl.when(c == 0)
      def _():
        pltpu.sync_copy(cnt_acc.at[pl.ds(r0, rows_per)], cnt_v)
        pltpu.sync_copy(cnt_v, cnt_out.at[pl.ds(r0, rows_per)])

  return agg


def _tc_layer(n, n_pad, d, hd, split_output):
  r = 2000
  dot = functools.partial(
      lax.dot_general,
      dimension_numbers=(((1,), (1,)), ((), ())),
      preferred_element_type=jnp.float32,
  )

  def body(x_ref, p_ref, c_ref, wl_ref, wr_ref, b_ref, o_ref):
    cnt = jnp.maximum(c_ref[...], 1.0)
    h = (dot(p_ref[0] / cnt, wl_ref[0]) + dot(p_ref[1] / cnt, wl_ref[1])
         + dot(x_ref[0], wr_ref[0]) + dot(x_ref[1], wr_ref[1])
         + b_ref[...])
    if split_output:
      h = jnp.maximum(h, 0.0)
      o_ref[0] = h[:, :hd]
      o_ref[1] = h[:, hd:]
    else:
      o_ref[...] = h

  if split_output:
    out_spec = pl.BlockSpec((NC, r, hd), lambda i: (0, i, 0))
    out_shape = jax.ShapeDtypeStruct((NC, n, hd), jnp.float32)
  else:
    out_spec = pl.BlockSpec((r, d), lambda i: (i, 0))
    out_shape = jax.ShapeDtypeStruct((n, d), jnp.float32)

  return pl.pallas_call(
      body,
      grid=(n // r,),
      in_specs=[
          pl.BlockSpec((NC, r, hd), lambda i: (0, i, 0)),
          pl.BlockSpec((NC, r, hd), lambda i: (0, i, 0)),
          pl.BlockSpec((r, 1), lambda i: (i, 0)),
          pl.BlockSpec((NC, d, hd), lambda i: (0, 0, 0)),
          pl.BlockSpec((NC, d, hd), lambda i: (0, 0, 0)),
          pl.BlockSpec((1, d), lambda i: (0, 0)),
      ],
      out_specs=out_spec,
      out_shape=out_shape,
  )


def kernel(x, edge_index, W1_l, b1, W1_r, W2_l, b2, W2_r):
  n, d = x.shape
  e = edge_index.shape[1]
  hd = d // 2

  k_chunks = -(-e // (NS * B * 2)) * 2
  e_pad = NS * B * k_chunks
  n_pad = -(-(n + 1) // (NS * 8)) * (NS * 8)

  src = edge_index[0]
  dst = edge_index[1]
  pad = e_pad - e
  src3 = jnp.concatenate([src, jnp.zeros((pad,), jnp.int32)]).reshape(
      NS, k_chunks, B)
  dst3 = jnp.concatenate([dst, jnp.full((pad,), n, jnp.int32)]).reshape(
      NS, k_chunks, B)

  z2 = jnp.zeros((n_pad, hd), jnp.float32)
  z1 = jnp.zeros((n_pad,), jnp.float32)
  ones = jnp.ones((B,), jnp.float32)

  agg1 = _sc_aggregate(n_pad, hd, k_chunks, with_counts=True)
  agg2 = _sc_aggregate(n_pad, hd, k_chunks, with_counts=False)
  l1 = _tc_layer(n, n_pad, d, hd, split_output=True)
  l2 = _tc_layer(n, n_pad, d, hd, split_output=False)

  x2 = jnp.stack([x[:, :hd], x[:, hd:]])
  w1l = jnp.stack([W1_l[:, :hd], W1_l[:, hd:]])
  w1r = jnp.stack([W1_r[:, :hd], W1_r[:, hd:]])
  w2l = jnp.stack([W2_l[:, :hd], W2_l[:, hd:]])
  w2r = jnp.stack([W2_r[:, :hd], W2_r[:, hd:]])
  b1r = b1.reshape(1, d)
  b2r = b2.reshape(1, d)

  p1, c1 = agg1(x2, src3, dst3, z2, z1, ones)
  c1r = c1.reshape(n_pad, 1)
  h2 = l1(x2, p1, c1r, w1l, w1r, b1r)
  p2, _ = agg2(h2, src3, dst3, z2, z1, ones)
  out = l2(h2, p2, c1r, w2l, w2r, b2r)
  return out


# speedup vs baseline: 2.4554x; 1.3009x over previous
import functools

import jax
import jax.numpy as jnp
from jax import lax
from jax.experimental import pallas as pl
from jax.experimental.pallas import tpu as pltpu
from jax.experimental.pallas import tpu_sc as plsc

NC = 2
NS = 16
B = 128


def _sc_aggregate(n, n_pad, hd, k_chunks, with_counts):
  rows_per = n_pad // NS
  tl_rows = n // NS

  mesh = plsc.VectorSubcoreMesh(core_axis_name="c", subcore_axis_name="s")

  @functools.partial(
      pl.kernel,
      mesh=mesh,
      compiler_params=pltpu.CompilerParams(use_tc_tiling_on_sc=False),
      out_type=[
          jax.ShapeDtypeStruct((NC, n_pad, hd), jnp.float32),
          jax.ShapeDtypeStruct((n_pad,), jnp.float32),
      ],
      scratch_types=[
          pltpu.VMEM((k_chunks, B), jnp.int32),
          pltpu.VMEM((4, B), jnp.int32),
          pltpu.VMEM((2, B, hd), jnp.float32),
          pltpu.VMEM((B,), jnp.float32),
          pltpu.VMEM((rows_per,), jnp.float32),
          pltpu.VMEM_SHARED((n, hd), jnp.float32),
          pltpu.VMEM_SHARED((n_pad, hd), jnp.float32),
          pltpu.VMEM_SHARED((n_pad,), jnp.float32),
          pltpu.SemaphoreType.DMA,
          pltpu.SemaphoreType.DMA,
          pltpu.SemaphoreType.DMA,
          pltpu.SemaphoreType.DMA,
          pltpu.SemaphoreType.DMA,
          pltpu.SemaphoreType.DMA,
          pltpu.SemaphoreType.DMA,
      ],
  )
  def agg(table_hbm, src_hbm, dst_hbm, z2_hbm, z1_hbm, ones_hbm,
          psum_out, cnt_out,
          src_all, dst_v, rows_v, ones_v, cnt_v, table_sp, accum, cnt_acc,
          gsem0, gsem1, ssem0, ssem1, isem0, isem1, csem):
    c = lax.axis_index("c")
    s = lax.axis_index("s")
    r0 = s * rows_per
    t0 = s * tl_rows
    gsem = (gsem0, gsem1)
    ssem = (ssem0, ssem1)
    isem = (isem0, isem1)
    half = table_hbm.at[c]

    pltpu.sync_copy(src_hbm.at[s], src_all)
    pltpu.sync_copy(half.at[pl.ds(t0, tl_rows)],
                    table_sp.at[pl.ds(t0, tl_rows)])
    pltpu.sync_copy(z2_hbm.at[pl.ds(r0, rows_per)],
                    accum.at[pl.ds(r0, rows_per)])
    if with_counts:
      @pl.when(c == 0)
      def _():
        pltpu.sync_copy(z1_hbm.at[pl.ds(r0, rows_per)], cnt_v)
        pltpu.sync_copy(cnt_v, cnt_acc.at[pl.ds(r0, rows_per)])
        pltpu.sync_copy(ones_hbm, ones_v)
    plsc.subcore_barrier()

    def fire_gather(k, par):
      pltpu.async_copy(table_sp.at[src_all.at[k]], rows_v.at[par],
                       gsem[par])

    def drain_gather(k, par):
      pltpu.make_async_copy(table_sp.at[src_all.at[k]], rows_v.at[par],
                            gsem[par]).wait()

    def fire_scatter(par, q):
      pltpu.async_copy(rows_v.at[par], accum.at[dst_v.at[q]],
                       ssem[par], add=True)

    def drain_scatter(par, q):
      pltpu.make_async_copy(rows_v.at[par], accum.at[dst_v.at[q]],
                            ssem[par]).wait()

    def fetch_dst(k, q, ip):
      pltpu.async_copy(dst_hbm.at[s, k], dst_v.at[q], isem[ip])

    def drain_dst(k, q, ip):
      pltpu.make_async_copy(dst_hbm.at[s, k], dst_v.at[q],
                            isem[ip]).wait()

    fetch_dst(0, 0, 0)
    drain_dst(0, 0, 0)
    fetch_dst(1, 1, 1)
    fire_gather(0, 0)

    def body(i4, carry):
      for q in range(4):
        k = i4 * 4 + q
        par = q % 2
        drain_gather(k, par)
        fire_scatter(par, q)
        if with_counts:
          @pl.when(c == 0)
          def _():
            pltpu.async_copy(ones_v, cnt_acc.at[dst_v.at[q]],
                             csem, add=True)

        @pl.when(k + 1 < k_chunks)
        def _():
          drain_dst(k + 1, (q + 1) % 4, (q + 1) % 2)
          @pl.when(k >= 1)
          def _():
            drain_scatter(1 - par, (q + 3) % 4)
          fire_gather(k + 1, 1 - par)

        @pl.when(k + 2 < k_chunks)
        def _():
          fetch_dst(k + 2, (q + 2) % 4, q % 2)

        if with_counts:
          @pl.when(c == 0)
          def _():
            pltpu.make_async_copy(ones_v, cnt_acc.at[dst_v.at[q]],
                                  csem).wait()
      return carry

    lax.fori_loop(0, k_chunks // 4, body, 0)
    drain_scatter((k_chunks - 2) % 2, (k_chunks - 2) % 4)
    drain_scatter((k_chunks - 1) % 2, (k_chunks - 1) % 4)
    plsc.subcore_barrier()

    pltpu.sync_copy(accum.at[pl.ds(r0, rows_per)],
                    psum_out.at[c, pl.ds(r0, rows_per)])
    if with_counts:
      @pl.when(c == 0)
      def _():
        pltpu.sync_copy(cnt_acc.at[pl.ds(r0, rows_per)], cnt_v)
        pltpu.sync_copy(cnt_v, cnt_out.at[pl.ds(r0, rows_per)])

  return agg


def _tc_layer(n, n_pad, d, hd, split_output):
  r = 2000
  dot = functools.partial(
      lax.dot_general,
      dimension_numbers=(((1,), (1,)), ((), ())),
      preferred_element_type=jnp.float32,
  )

  def body(x_ref, p_ref, c_ref, wl_ref, wr_ref, b_ref, o_ref):
    cnt = jnp.maximum(c_ref[...], 1.0)
    h = (dot(p_ref[0] / cnt, wl_ref[0]) + dot(p_ref[1] / cnt, wl_ref[1])
         + dot(x_ref[0], wr_ref[0]) + dot(x_ref[1], wr_ref[1])
         + b_ref[...])
    if split_output:
      h = jnp.maximum(h, 0.0)
      o_ref[0] = h[:, :hd]
      o_ref[1] = h[:, hd:]
    else:
      o_ref[...] = h

  if split_output:
    out_spec = pl.BlockSpec((NC, r, hd), lambda i: (0, i, 0))
    out_shape = jax.ShapeDtypeStruct((NC, n, hd), jnp.float32)
  else:
    out_spec = pl.BlockSpec((r, d), lambda i: (i, 0))
    out_shape = jax.ShapeDtypeStruct((n, d), jnp.float32)

  return pl.pallas_call(
      body,
      grid=(n // r,),
      in_specs=[
          pl.BlockSpec((NC, r, hd), lambda i: (0, i, 0)),
          pl.BlockSpec((NC, r, hd), lambda i: (0, i, 0)),
          pl.BlockSpec((r, 1), lambda i: (i, 0)),
          pl.BlockSpec((NC, d, hd), lambda i: (0, 0, 0)),
          pl.BlockSpec((NC, d, hd), lambda i: (0, 0, 0)),
          pl.BlockSpec((1, d), lambda i: (0, 0)),
      ],
      out_specs=out_spec,
      out_shape=out_shape,
  )


def kernel(x, edge_index, W1_l, b1, W1_r, W2_l, b2, W2_r):
  n, d = x.shape
  e = edge_index.shape[1]
  hd = d // 2

  k_chunks = -(-e // (NS * B * 4)) * 4
  e_pad = NS * B * k_chunks
  n_pad = -(-(n + 1) // (NS * 8)) * (NS * 8)

  src = edge_index[0]
  dst = edge_index[1]
  pad = e_pad - e
  src3 = jnp.concatenate([src, jnp.zeros((pad,), jnp.int32)]).reshape(
      NS, k_chunks, B)
  dst3 = jnp.concatenate([dst, jnp.full((pad,), n, jnp.int32)]).reshape(
      NS, k_chunks, B)

  z2 = jnp.zeros((n_pad, hd), jnp.float32)
  z1 = jnp.zeros((n_pad,), jnp.float32)
  ones = jnp.ones((B,), jnp.float32)

  agg1 = _sc_aggregate(n, n_pad, hd, k_chunks, with_counts=True)
  agg2 = _sc_aggregate(n, n_pad, hd, k_chunks, with_counts=False)
  l1 = _tc_layer(n, n_pad, d, hd, split_output=True)
  l2 = _tc_layer(n, n_pad, d, hd, split_output=False)

  x2 = jnp.stack([x[:, :hd], x[:, hd:]])
  w1l = jnp.stack([W1_l[:, :hd], W1_l[:, hd:]])
  w1r = jnp.stack([W1_r[:, :hd], W1_r[:, hd:]])
  w2l = jnp.stack([W2_l[:, :hd], W2_l[:, hd:]])
  w2r = jnp.stack([W2_r[:, :hd], W2_r[:, hd:]])
  b1r = b1.reshape(1, d)
  b2r = b2.reshape(1, d)

  p1, c1 = agg1(x2, src3, dst3, z2, z1, ones)
  c1r = c1.reshape(n_pad, 1)
  h2 = l1(x2, p1, c1r, w1l, w1r, b1r)
  p2, _ = agg2(h2, src3, dst3, z2, z1, ones)
  out = l2(h2, p2, c1r, w2l, w2r, b2r)
  return out


# speedup vs baseline: 2.5776x; 1.0498x over previous
import functools

import jax
import jax.numpy as jnp
from jax import lax
from jax.experimental import pallas as pl
from jax.experimental.pallas import tpu as pltpu
from jax.experimental.pallas import tpu_sc as plsc

NC = 2
NS = 16
B = 128


def _sc_aggregate(n, n_pad, hd, k_chunks, with_counts):
  rows_per = n_pad // NS
  tl_rows = n // NS

  mesh = plsc.VectorSubcoreMesh(core_axis_name="c", subcore_axis_name="s")

  @functools.partial(
      pl.kernel,
      mesh=mesh,
      compiler_params=pltpu.CompilerParams(use_tc_tiling_on_sc=False),
      out_type=[
          jax.ShapeDtypeStruct((NC, n_pad, hd), jnp.float32),
          jax.ShapeDtypeStruct((NC * n_pad,), jnp.float32),
      ],
      scratch_types=[
          pltpu.VMEM((k_chunks, B), jnp.int32),
          pltpu.VMEM((4, B), jnp.int32),
          pltpu.VMEM((2, B, hd), jnp.float32),
          pltpu.VMEM((B,), jnp.float32),
          pltpu.VMEM((rows_per,), jnp.float32),
          pltpu.VMEM_SHARED((n, hd), jnp.float32),
          pltpu.VMEM_SHARED((n_pad, hd), jnp.float32),
          pltpu.VMEM_SHARED((n_pad,), jnp.float32),
          pltpu.SemaphoreType.DMA,
          pltpu.SemaphoreType.DMA,
          pltpu.SemaphoreType.DMA,
          pltpu.SemaphoreType.DMA,
          pltpu.SemaphoreType.DMA,
          pltpu.SemaphoreType.DMA,
          pltpu.SemaphoreType.DMA,
      ],
  )
  def agg(table_hbm, src_hbm, dst_hbm, z2_hbm, z1_hbm, ones_hbm,
          psum_out, cnt_out,
          src_all, dst_v, rows_v, ones_v, cnt_v, table_sp, accum, cnt_acc,
          gsem0, gsem1, ssem0, ssem1, isem0, isem1, csem):
    c = lax.axis_index("c")
    s = lax.axis_index("s")
    r0 = s * rows_per
    t0 = s * tl_rows
    gsem = (gsem0, gsem1)
    ssem = (ssem0, ssem1)
    isem = (isem0, isem1)
    half = table_hbm.at[c]

    pltpu.sync_copy(src_hbm.at[s], src_all)
    pltpu.sync_copy(half.at[pl.ds(t0, tl_rows)],
                    table_sp.at[pl.ds(t0, tl_rows)])
    pltpu.sync_copy(z2_hbm.at[pl.ds(r0, rows_per)],
                    accum.at[pl.ds(r0, rows_per)])
    if with_counts:
      pltpu.sync_copy(z1_hbm.at[pl.ds(r0, rows_per)], cnt_v)
      pltpu.sync_copy(cnt_v, cnt_acc.at[pl.ds(r0, rows_per)])
      pltpu.sync_copy(ones_hbm, ones_v)
    plsc.subcore_barrier()

    def fire_gather(k, par):
      pltpu.async_copy(table_sp.at[src_all.at[k]], rows_v.at[par],
                       gsem[par])

    def drain_gather(k, par):
      pltpu.make_async_copy(table_sp.at[src_all.at[k]], rows_v.at[par],
                            gsem[par]).wait()

    def fire_scatter(par, q):
      pltpu.async_copy(rows_v.at[par], accum.at[dst_v.at[q]],
                       ssem[par], add=True)

    def drain_scatter(par, q):
      pltpu.make_async_copy(rows_v.at[par], accum.at[dst_v.at[q]],
                            ssem[par]).wait()

    def fetch_dst(k, q, ip):
      pltpu.async_copy(dst_hbm.at[s, k], dst_v.at[q], isem[ip])

    def drain_dst(k, q, ip):
      pltpu.make_async_copy(dst_hbm.at[s, k], dst_v.at[q],
                            isem[ip]).wait()

    fetch_dst(0, 0, 0)
    drain_dst(0, 0, 0)
    fetch_dst(1, 1, 1)
    fire_gather(0, 0)

    def body(i4, carry):
      for q in range(4):
        k = i4 * 4 + q
        par = q % 2
        drain_gather(k, par)
        fire_scatter(par, q)
        if with_counts:
          @pl.when(c == par)
          def _():
            pltpu.async_copy(ones_v, cnt_acc.at[dst_v.at[q]],
                             csem, add=True)

        @pl.when(k + 1 < k_chunks)
        def _():
          drain_dst(k + 1, (q + 1) % 4, (q + 1) % 2)
          @pl.when(k >= 1)
          def _():
            drain_scatter(1 - par, (q + 3) % 4)
          fire_gather(k + 1, 1 - par)

        @pl.when(k + 2 < k_chunks)
        def _():
          fetch_dst(k + 2, (q + 2) % 4, q % 2)

        if with_counts:
          @pl.when(c == par)
          def _():
            pltpu.make_async_copy(ones_v, cnt_acc.at[dst_v.at[q]],
                                  csem).wait()
      return carry

    lax.fori_loop(0, k_chunks // 4, body, 0)
    drain_scatter((k_chunks - 2) % 2, (k_chunks - 2) % 4)
    drain_scatter((k_chunks - 1) % 2, (k_chunks - 1) % 4)
    plsc.subcore_barrier()

    pltpu.sync_copy(accum.at[pl.ds(r0, rows_per)],
                    psum_out.at[c, pl.ds(r0, rows_per)])
    if with_counts:
      pltpu.sync_copy(cnt_acc.at[pl.ds(r0, rows_per)], cnt_v)
      pltpu.sync_copy(cnt_v, cnt_out.at[pl.ds(c * n_pad + r0, rows_per)])

  return agg


def _tc_layer(n, n_pad, d, hd, split_output):
  r = 2000
  dot = functools.partial(
      lax.dot_general,
      dimension_numbers=(((1,), (1,)), ((), ())),
      preferred_element_type=jnp.float32,
  )

  def body(x_ref, p_ref, c_ref, wl_ref, wr_ref, b_ref, o_ref):
    cnt = jnp.maximum(c_ref[0] + c_ref[1], 1.0)
    h = (dot(p_ref[0] / cnt, wl_ref[0]) + dot(p_ref[1] / cnt, wl_ref[1])
         + dot(x_ref[0], wr_ref[0]) + dot(x_ref[1], wr_ref[1])
         + b_ref[...])
    if split_output:
      h = jnp.maximum(h, 0.0)
      o_ref[0] = h[:, :hd]
      o_ref[1] = h[:, hd:]
    else:
      o_ref[...] = h

  if split_output:
    out_spec = pl.BlockSpec((NC, r, hd), lambda i: (0, i, 0))
    out_shape = jax.ShapeDtypeStruct((NC, n, hd), jnp.float32)
  else:
    out_spec = pl.BlockSpec((r, d), lambda i: (i, 0))
    out_shape = jax.ShapeDtypeStruct((n, d), jnp.float32)

  return pl.pallas_call(
      body,
      grid=(n // r,),
      in_specs=[
          pl.BlockSpec((NC, r, hd), lambda i: (0, i, 0)),
          pl.BlockSpec((NC, r, hd), lambda i: (0, i, 0)),
          pl.BlockSpec((NC, r, 1), lambda i: (0, i, 0)),
          pl.BlockSpec((NC, d, hd), lambda i: (0, 0, 0)),
          pl.BlockSpec((NC, d, hd), lambda i: (0, 0, 0)),
          pl.BlockSpec((1, d), lambda i: (0, 0)),
      ],
      out_specs=out_spec,
      out_shape=out_shape,
  )


def kernel(x, edge_index, W1_l, b1, W1_r, W2_l, b2, W2_r):
  n, d = x.shape
  e = edge_index.shape[1]
  hd = d // 2

  k_chunks = -(-e // (NS * B * 4)) * 4
  e_pad = NS * B * k_chunks
  n_pad = -(-(n + 1) // (NS * 8)) * (NS * 8)

  src = edge_index[0]
  dst = edge_index[1]
  pad = e_pad - e
  src3 = jnp.concatenate([src, jnp.zeros((pad,), jnp.int32)]).reshape(
      NS, k_chunks, B)
  dst3 = jnp.concatenate([dst, jnp.full((pad,), n, jnp.int32)]).reshape(
      NS, k_chunks, B)

  z2 = jnp.zeros((n_pad, hd), jnp.float32)
  z1 = jnp.zeros((n_pad,), jnp.float32)
  ones = jnp.ones((B,), jnp.float32)

  agg1 = _sc_aggregate(n, n_pad, hd, k_chunks, with_counts=True)
  agg2 = _sc_aggregate(n, n_pad, hd, k_chunks, with_counts=False)
  l1 = _tc_layer(n, n_pad, d, hd, split_output=True)
  l2 = _tc_layer(n, n_pad, d, hd, split_output=False)

  x2 = jnp.stack([x[:, :hd], x[:, hd:]])
  w1l = jnp.stack([W1_l[:, :hd], W1_l[:, hd:]])
  w1r = jnp.stack([W1_r[:, :hd], W1_r[:, hd:]])
  w2l = jnp.stack([W2_l[:, :hd], W2_l[:, hd:]])
  w2r = jnp.stack([W2_r[:, :hd], W2_r[:, hd:]])
  b1r = b1.reshape(1, d)
  b2r = b2.reshape(1, d)

  p1, c1 = agg1(x2, src3, dst3, z2, z1, ones)
  c1r = c1.reshape(NC, n_pad, 1)
  h2 = l1(x2, p1, c1r, w1l, w1r, b1r)
  p2, _ = agg2(h2, src3, dst3, z2, z1, ones)
  out = l2(h2, p2, c1r, w2l, w2r, b2r)
  return out
